# Initial kernel scaffold; baseline (speedup 1.0000x reference)
import jax
import jax.numpy as jnp
from jax.experimental import pallas as pl


def kernel(x, edge_index, W1, b1, W2, b2, W3, b3, W4, b4):
    raise NotImplementedError("write your pallas kernel here")


# speedup vs baseline: 13.2535x; 13.2535x over previous
import jax
import jax.numpy as jnp
from jax import lax
from jax.experimental import pallas as pl
from jax.experimental.pallas import tpu as pltpu
from jax.experimental.pallas import tpu_sc as plsc

N = 10000
NP = 10240
E = 320000
NC = 2
NS = 16
NW = NC * NS
EPW = E // NW
CH = 125
ROWS_PER_W = EPW // CH
GRP = 16
NG = ROWS_PER_W // GRP
RW = 16

_mesh = plsc.VectorSubcoreMesh(core_axis_name="c", subcore_axis_name="s")
_sc_params = pltpu.CompilerParams(use_tc_tiling_on_sc=False)


def _degrees_body(ei3, do_out, di_out, sbuf, dbuf, obuf, zc, ssem, dsem, zsem,
                  sh_do, sh_di):
    c = lax.axis_index("c")
    s = lax.axis_index("s")
    wid = s * NC + c
    ones16 = jnp.full((16,), 1.0, jnp.float32)
    zv = jnp.zeros((16,), jnp.float32)

    def fill(i, _):
        obuf[i, :] = ones16
        return 0

    lax.fori_loop(0, CH, fill, 0)
    for r in range(16):
        zc[r, :] = zv

    def zbatch(i, _):
        for t in range(4):
            g = (s * 40 + i * 4 + t) * 16
            pltpu.async_copy(zc, sh_do.at[pl.ds(g, 16)], zsem)
            pltpu.async_copy(zc, sh_di.at[pl.ds(g, 16)], zsem)
        for t in range(4):
            g = (s * 40 + i * 4 + t) * 16
            pltpu.make_async_copy(zc, sh_do.at[pl.ds(g, 16)], zsem).wait()
            pltpu.make_async_copy(zc, sh_di.at[pl.ds(g, 16)], zsem).wait()
        return 0

    lax.fori_loop(0, (NP // 16) // NS // 4, zbatch, 0)
    plsc.subcore_barrier()

    row0 = wid * ROWS_PER_W
    for g in range(NG):
        pltpu.sync_copy(ei3.at[0, pl.ds(row0 + g * GRP, GRP)], sbuf)
        pltpu.sync_copy(ei3.at[1, pl.ds(row0 + g * GRP, GRP)], dbuf)

        def batch(i, _):
            for t in range(4):
                j = i * 4 + t
                pltpu.async_copy(obuf, sh_do.at[sbuf.at[j]], ssem, add=True)
                pltpu.async_copy(obuf, sh_di.at[dbuf.at[j]], dsem, add=True)
            for t in range(4):
                j = i * 4 + t
                pltpu.make_async_copy(obuf, sh_do.at[sbuf.at[j]],
                                      ssem).wait()
                pltpu.make_async_copy(obuf, sh_di.at[dbuf.at[j]],
                                      dsem).wait()
            return 0

        lax.fori_loop(0, GRP // 4, batch, 0)
    plsc.subcore_barrier()

    @pl.when(s == 0)
    def _():
        pltpu.sync_copy(sh_do, do_out.at[c])
        pltpu.sync_copy(sh_di, di_out.at[c])


_DEG_OUT_TYPE = (
    jax.ShapeDtypeStruct((NC, NP, RW), jnp.float32),
    jax.ShapeDtypeStruct((NC, NP, RW), jnp.float32),
)
_DEG_SCRATCH = (
    pltpu.VMEM((GRP, CH), jnp.int32),
    pltpu.VMEM((GRP, CH), jnp.int32),
    pltpu.VMEM((CH, RW), jnp.float32),
    pltpu.VMEM((16, RW), jnp.float32),
    pltpu.SemaphoreType.DMA,
    pltpu.SemaphoreType.DMA,
    pltpu.SemaphoreType.DMA,
    pltpu.VMEM_SHARED((NP, RW), jnp.float32),
    pltpu.VMEM_SHARED((NP, RW), jnp.float32),
)
_degrees_kernel = pl.kernel(
    _degrees_body, out_type=_DEG_OUT_TYPE, mesh=_mesh,
    scratch_types=_DEG_SCRATCH, compiler_params=_sc_params)


def _make_agg_parts(F, with_c, NBUF):
    out_type = [jax.ShapeDtypeStruct((NC, NP, F), jnp.float32)]
    scratch = (
        [pltpu.VMEM((GRP, CH), jnp.int32),
         pltpu.VMEM((GRP, CH), jnp.int32)]
        + [pltpu.VMEM((CH, F), jnp.float32)] * NBUF
        + [pltpu.VMEM((16, F), jnp.float32)]
        + [pltpu.SemaphoreType.DMA] * (2 * NBUF + 1)
        + [pltpu.VMEM_SHARED((NP, F), jnp.float32)]
    )
    if with_c:
        out_type.append(jax.ShapeDtypeStruct((NC, NP, RW), jnp.float32))
        scratch += (
            [pltpu.VMEM((CH, RW), jnp.float32)] * NBUF
            + [pltpu.VMEM((16, RW), jnp.float32)]
            + [pltpu.SemaphoreType.DMA] * (2 * NBUF)
            + [pltpu.VMEM_SHARED((NP, RW), jnp.float32)]
        )

    def body(*args):
        if with_c:
            (h, ei3, dii, agg_out, c_out) = args[:5]
            rest = args[5:]
        else:
            (h, ei3, agg_out) = args[:3]
            rest = args[3:]
        sbuf, dbuf = rest[0], rest[1]
        rows = rest[2:2 + NBUF]
        zbuf = rest[2 + NBUF]
        gsem = rest[3 + NBUF:3 + 2 * NBUF]
        ssem = rest[3 + 2 * NBUF:3 + 3 * NBUF]
        zsem = rest[3 + 3 * NBUF]
        sh_agg = rest[4 + 3 * NBUF]
        if with_c:
            crest = rest[5 + 3 * NBUF:]
            vals = crest[:NBUF]
            zc = crest[NBUF]
            vgsem = crest[NBUF + 1:2 * NBUF + 1]
            vssem = crest[2 * NBUF + 1:3 * NBUF + 1]
            sh_c = crest[3 * NBUF + 1]
        c = lax.axis_index("c")
        s = lax.axis_index("s")
        wid = s * NC + c
        zv = jnp.zeros((16,), jnp.float32)
        for r in range(16):
            for q in range(F // 16):
                zbuf[r, pl.ds(q * 16, 16)] = zv
        if with_c:
            for r in range(16):
                zc[r, :] = zv

        def zbatch(i, _):
            for t in range(4):
                g = (s * 40 + i * 4 + t) * 16
                pltpu.async_copy(zbuf, sh_agg.at[pl.ds(g, 16)], zsem)
                if with_c:
                    pltpu.async_copy(zc, sh_c.at[pl.ds(g, 16)], zsem)
            for t in range(4):
                g = (s * 40 + i * 4 + t) * 16
                pltpu.make_async_copy(zbuf, sh_agg.at[pl.ds(g, 16)],
                                      zsem).wait()
                if with_c:
                    pltpu.make_async_copy(zc, sh_c.at[pl.ds(g, 16)],
                                          zsem).wait()
            return 0

        lax.fori_loop(0, (NP // 16) // NS // 4, zbatch, 0)
        plsc.subcore_barrier()

        def gather(j, b):
            pltpu.async_copy(h.at[sbuf.at[j]], rows[b], gsem[b])
            if with_c:
                pltpu.async_copy(dii.at[dbuf.at[j]], vals[b], vgsem[b])

        def wait_gather(j, b):
            pltpu.make_async_copy(h.at[sbuf.at[j]], rows[b], gsem[b]).wait()
            if with_c:
                pltpu.make_async_copy(dii.at[dbuf.at[j]], vals[b],
                                      vgsem[b]).wait()

        def scatter(j, b):
            pltpu.async_copy(rows[b], sh_agg.at[dbuf.at[j]], ssem[b],
                             add=True)
            if with_c:
                pltpu.async_copy(vals[b], sh_c.at[sbuf.at[j]], vssem[b],
                                 add=True)

        def wait_scatter(j, b):
            pltpu.make_async_copy(rows[b], sh_agg.at[dbuf.at[j]],
                                  ssem[b]).wait()
            if with_c:
                pltpu.make_async_copy(vals[b], sh_c.at[sbuf.at[j]],
                                      vssem[b]).wait()

        row0 = wid * ROWS_PER_W
        for g in range(NG):
            pltpu.sync_copy(ei3.at[0, pl.ds(row0 + g * GRP, GRP)], sbuf)
            pltpu.sync_copy(ei3.at[1, pl.ds(row0 + g * GRP, GRP)], dbuf)
            gather(0, 0)

            def quad(i, _):
                for t in range(NBUF):
                    j = i * NBUF + t
                    wait_gather(j, t)
                    scatter(j, t)
                    bn = (t + 1) % NBUF

                    @pl.when(j + 1 < GRP)
                    def _():
                        @pl.when(j - (NBUF - 1) >= 0)
                        def _():
                            wait_scatter(j - (NBUF - 1), bn)
                        gather(j + 1, bn)
                return 0

            lax.fori_loop(0, GRP // NBUF, quad, 0)
            for t in range(NBUF):
                wait_scatter(GRP - NBUF + t, t)
        plsc.subcore_barrier()
        rpt = NP // NS
        pltpu.sync_copy(sh_agg.at[pl.ds(s * rpt, rpt)],
                        agg_out.at[c, pl.ds(s * rpt, rpt)])
        if with_c:
            @pl.when(s == 0)
            def _():
                pltpu.sync_copy(sh_c, c_out.at[c])

    return body, tuple(out_type), tuple(scratch)


def _make_agg_kernel(F, with_c, nbuf):
    body, out_type, scratch = _make_agg_parts(F, with_c, nbuf)
    return pl.kernel(body, out_type=out_type, mesh=_mesh,
                     scratch_types=scratch, compiler_params=_sc_params)


_agg1_kernel = _make_agg_kernel(128, with_c=False, nbuf=2)
_agg2_kernel = _make_agg_kernel(64, with_c=False, nbuf=4)
_agg3_kernel = _make_agg_kernel(32, with_c=True, nbuf=4)


def _first_body(x_ref, w_ref, dop_ref, dip_ref, h_ref, dio_ref, dii_ref,
                dii16_ref):
    deg_o = jnp.maximum(dop_ref[0, :N, :1] + dop_ref[1, :N, :1], 1.0)
    deg_i = jnp.maximum(dip_ref[0, :N, :1] + dip_ref[1, :N, :1], 1.0)
    dio = lax.rsqrt(deg_o)
    dii = lax.rsqrt(deg_i)
    dio_ref[...] = dio
    dii_ref[...] = dii
    dii16_ref[...] = jnp.broadcast_to(dii, (N, RW))
    h_ref[...] = jnp.dot(x_ref[...] * dio, w_ref[...],
                         preferred_element_type=jnp.float32)


_first_call = pl.pallas_call(
    _first_body,
    out_shape=(
        jax.ShapeDtypeStruct((N, 128), jnp.float32),
        jax.ShapeDtypeStruct((N, 1), jnp.float32),
        jax.ShapeDtypeStruct((N, 1), jnp.float32),
        jax.ShapeDtypeStruct((N, RW), jnp.float32),
    ),
)


def _mid_body(aggp_ref, dii_ref, dio_ref, b_ref, w_ref, out_ref):
    h = (aggp_ref[0, :N] + aggp_ref[1, :N]) * dii_ref[...] + b_ref[...]
    h = jnp.maximum(h, 0.0) * dio_ref[...]
    out_ref[...] = jnp.dot(h, w_ref[...], preferred_element_type=jnp.float32)


def _mid_call(fout):
    return pl.pallas_call(
        _mid_body,
        out_shape=jax.ShapeDtypeStruct((N, fout), jnp.float32),
    )


def _final_body(aggp_ref, dii_ref, dio_ref, b3_ref, cp_ref, w4_ref, b4_ref,
                out_ref):
    x4 = (aggp_ref[0, :N] + aggp_ref[1, :N]) * dii_ref[...] + b3_ref[...]
    x4 = jnp.maximum(x4, 0.0) * dio_ref[...]
    w = cp_ref[0, :N, :1] + cp_ref[1, :N, :1]
    u = jnp.sum(x4 * w, axis=0, keepdims=True)
    out_ref[...] = (jnp.dot(u, w4_ref[...], preferred_element_type=jnp.float32)
                    * (1.0 / N) + b4_ref[...])


_final_call = pl.pallas_call(
    _final_body,
    out_shape=jax.ShapeDtypeStruct((1, 32), jnp.float32),
)


@jax.jit
def kernel(x, edge_index, W1, b1, W2, b2, W3, b3, W4, b4):
    ei3 = edge_index.astype(jnp.int32).reshape(2, E // CH, CH)

    dop, dip = _degrees_kernel(ei3)
    h1, dio, dii, dii16 = _first_call(x, W1, dop, dip)
    agg1, = _agg1_kernel(h1, ei3)
    h2 = _mid_call(64)(agg1, dii, dio, b1.reshape(1, -1), W2)
    agg2, = _agg2_kernel(h2, ei3)
    h3 = _mid_call(32)(agg2, dii, dio, b2.reshape(1, -1), W3)
    agg3, cp = _agg3_kernel(h3, ei3, dii16)
    out = _final_call(agg3, dii, dio, b3.reshape(1, -1),
                      cp, W4, b4.reshape(1, -1))
    return out


# speedup vs baseline: 15.9354x; 1.2024x over previous
import jax
import jax.numpy as jnp
from jax import lax
from jax.experimental import pallas as pl
from jax.experimental.pallas import tpu as pltpu
from jax.experimental.pallas import tpu_sc as plsc

N = 10000
NP = 10240
E = 320000
NC = 2
NS = 16
NW = NC * NS
EPW = E // NW
CH = 125
ROWS_PER_W = EPW // CH
GRP = 16
NG = ROWS_PER_W // GRP
AGRP = 40
ANG = ROWS_PER_W // AGRP
RW = 16

_mesh = plsc.VectorSubcoreMesh(core_axis_name="c", subcore_axis_name="s")
_sc_params = pltpu.CompilerParams(use_tc_tiling_on_sc=False)


def _degrees_body(ei3, do_out, di_out, sbuf, dbuf, obuf, zc, ssem, dsem, zsem,
                  sh_do, sh_di):
    c = lax.axis_index("c")
    s = lax.axis_index("s")
    wid = s * NC + c
    ones16 = jnp.full((16,), 1.0, jnp.float32)
    zv = jnp.zeros((16,), jnp.float32)

    def fill(i, _):
        obuf[i, :] = ones16
        return 0

    lax.fori_loop(0, CH, fill, 0)
    for r in range(16):
        zc[r, :] = zv

    def zbatch(i, _):
        for t in range(4):
            g = (s * 40 + i * 4 + t) * 16
            pltpu.async_copy(zc, sh_do.at[pl.ds(g, 16)], zsem)
            pltpu.async_copy(zc, sh_di.at[pl.ds(g, 16)], zsem)
        for t in range(4):
            g = (s * 40 + i * 4 + t) * 16
            pltpu.make_async_copy(zc, sh_do.at[pl.ds(g, 16)], zsem).wait()
            pltpu.make_async_copy(zc, sh_di.at[pl.ds(g, 16)], zsem).wait()
        return 0

    lax.fori_loop(0, (NP // 16) // NS // 4, zbatch, 0)
    plsc.subcore_barrier()

    row0 = wid * ROWS_PER_W
    for g in range(NG):
        pltpu.sync_copy(ei3.at[0, pl.ds(row0 + g * GRP, GRP)], sbuf)
        pltpu.sync_copy(ei3.at[1, pl.ds(row0 + g * GRP, GRP)], dbuf)

        def batch(i, _):
            for t in range(4):
                j = i * 4 + t
                pltpu.async_copy(obuf, sh_do.at[sbuf.at[j]], ssem, add=True)
                pltpu.async_copy(obuf, sh_di.at[dbuf.at[j]], dsem, add=True)
            for t in range(4):
                j = i * 4 + t
                pltpu.make_async_copy(obuf, sh_do.at[sbuf.at[j]],
                                      ssem).wait()
                pltpu.make_async_copy(obuf, sh_di.at[dbuf.at[j]],
                                      dsem).wait()
            return 0

        lax.fori_loop(0, GRP // 4, batch, 0)
    plsc.subcore_barrier()

    @pl.when(s == 0)
    def _():
        pltpu.sync_copy(sh_do, do_out.at[c])
        pltpu.sync_copy(sh_di, di_out.at[c])


_DEG_OUT_TYPE = (
    jax.ShapeDtypeStruct((NC, NP, RW), jnp.float32),
    jax.ShapeDtypeStruct((NC, NP, RW), jnp.float32),
)
_DEG_SCRATCH = (
    pltpu.VMEM((GRP, CH), jnp.int32),
    pltpu.VMEM((GRP, CH), jnp.int32),
    pltpu.VMEM((CH, RW), jnp.float32),
    pltpu.VMEM((16, RW), jnp.float32),
    pltpu.SemaphoreType.DMA,
    pltpu.SemaphoreType.DMA,
    pltpu.SemaphoreType.DMA,
    pltpu.VMEM_SHARED((NP, RW), jnp.float32),
    pltpu.VMEM_SHARED((NP, RW), jnp.float32),
)
_degrees_kernel = pl.kernel(
    _degrees_body, out_type=_DEG_OUT_TYPE, mesh=_mesh,
    scratch_types=_DEG_SCRATCH, compiler_params=_sc_params)


def _make_agg_parts(F, with_c, NBUF=2):
    out_type = [jax.ShapeDtypeStruct((NC, NP, F), jnp.float32)]
    scratch = [
        pltpu.VMEM((AGRP, CH), jnp.int32),
        pltpu.VMEM((AGRP, CH), jnp.int32),
        pltpu.VMEM((CH, F), jnp.float32),
        pltpu.VMEM((CH, F), jnp.float32),
        pltpu.VMEM((16, F), jnp.float32),
        pltpu.SemaphoreType.DMA,
        pltpu.SemaphoreType.DMA,
        pltpu.SemaphoreType.DMA,
        pltpu.VMEM_SHARED((NP, F), jnp.float32),
    ]
    if with_c:
        out_type.append(jax.ShapeDtypeStruct((NC, NP, RW), jnp.float32))
        scratch += [
            pltpu.VMEM((CH, RW), jnp.float32),
            pltpu.VMEM((CH, RW), jnp.float32),
            pltpu.VMEM((16, RW), jnp.float32),
            pltpu.SemaphoreType.DMA,
            pltpu.SemaphoreType.DMA,
            pltpu.VMEM_SHARED((NP, RW), jnp.float32),
        ]

    def body(*args):
        if with_c:
            (h, ei3, dii, agg_out, c_out,
             sbuf, dbuf, rows0, rows1, zbuf, sem0, sem1, zsem, sh_agg,
             vals0, vals1, zc, vsem0, vsem1, sh_c) = args
            vals = (vals0, vals1)
            vsem = (vsem0, vsem1)
        else:
            (h, ei3, agg_out,
             sbuf, dbuf, rows0, rows1, zbuf, sem0, sem1, zsem, sh_agg) = args
        rows = (rows0, rows1)
        sem = (sem0, sem1)
        c = lax.axis_index("c")
        s = lax.axis_index("s")
        wid = s * NC + c
        zv = jnp.zeros((16,), jnp.float32)
        for r in range(16):
            for q in range(F // 16):
                zbuf[r, pl.ds(q * 16, 16)] = zv
        if with_c:
            for r in range(16):
                zc[r, :] = zv

        def zbatch(i, _):
            for t in range(4):
                g = (s * 40 + i * 4 + t) * 16
                pltpu.async_copy(zbuf, sh_agg.at[pl.ds(g, 16)], zsem)
                if with_c:
                    pltpu.async_copy(zc, sh_c.at[pl.ds(g, 16)], zsem)
            for t in range(4):
                g = (s * 40 + i * 4 + t) * 16
                pltpu.make_async_copy(zbuf, sh_agg.at[pl.ds(g, 16)],
                                      zsem).wait()
                if with_c:
                    pltpu.make_async_copy(zc, sh_c.at[pl.ds(g, 16)],
                                          zsem).wait()
            return 0

        lax.fori_loop(0, (NP // 16) // NS // 4, zbatch, 0)
        plsc.subcore_barrier()

        def gathers(j, b):
            pltpu.async_copy(h.at[sbuf.at[j]], rows[b], sem[b])
            if with_c:
                pltpu.async_copy(dii.at[dbuf.at[j]], vals[b], vsem[b])

        def drain(j, b):
            pltpu.make_async_copy(h.at[sbuf.at[j]], rows[b], sem[b]).wait()
            pltpu.sync_copy(rows[b], sh_agg.at[dbuf.at[j]], add=True)
            if with_c:
                pltpu.make_async_copy(dii.at[dbuf.at[j]], vals[b],
                                      vsem[b]).wait()
                pltpu.sync_copy(vals[b], sh_c.at[sbuf.at[j]], add=True)

        def chunk2(i, _):
            j0 = 2 * i
            gathers(j0 + 1, 1)
            drain(j0, 0)

            @pl.when(j0 + 2 < AGRP)
            def _():
                gathers(j0 + 2, 0)

            drain(j0 + 1, 1)
            return 0

        row0 = wid * ROWS_PER_W
        for g in range(ANG):
            pltpu.sync_copy(ei3.at[0, pl.ds(row0 + g * AGRP, AGRP)], sbuf)
            pltpu.sync_copy(ei3.at[1, pl.ds(row0 + g * AGRP, AGRP)], dbuf)
            gathers(0, 0)
            lax.fori_loop(0, AGRP // 2, chunk2, 0)
        plsc.subcore_barrier()
        rpt = NP // NS
        pltpu.sync_copy(sh_agg.at[pl.ds(s * rpt, rpt)],
                        agg_out.at[c, pl.ds(s * rpt, rpt)])
        if with_c:
            @pl.when(s == 0)
            def _():
                pltpu.sync_copy(sh_c, c_out.at[c])

    return body, tuple(out_type), tuple(scratch)


def _make_agg_kernel(F, with_c):
    body, out_type, scratch = _make_agg_parts(F, with_c)
    return pl.kernel(body, out_type=out_type, mesh=_mesh,
                     scratch_types=scratch, compiler_params=_sc_params)


_agg1_kernel = _make_agg_kernel(128, with_c=False)
_agg2_kernel = _make_agg_kernel(64, with_c=False)
_agg3_kernel = _make_agg_kernel(32, with_c=True)


def _first_body(x_ref, w_ref, dop_ref, dip_ref, h_ref, dio_ref, dii_ref,
                dii16_ref):
    deg_o = jnp.maximum(dop_ref[0, :N, :1] + dop_ref[1, :N, :1], 1.0)
    deg_i = jnp.maximum(dip_ref[0, :N, :1] + dip_ref[1, :N, :1], 1.0)
    dio = lax.rsqrt(deg_o)
    dii = lax.rsqrt(deg_i)
    dio_ref[...] = dio
    dii_ref[...] = dii
    dii16_ref[...] = jnp.broadcast_to(dii, (N, RW))
    h_ref[...] = jnp.dot(x_ref[...] * dio, w_ref[...],
                         preferred_element_type=jnp.float32)


_first_call = pl.pallas_call(
    _first_body,
    out_shape=(
        jax.ShapeDtypeStruct((N, 128), jnp.float32),
        jax.ShapeDtypeStruct((N, 1), jnp.float32),
        jax.ShapeDtypeStruct((N, 1), jnp.float32),
        jax.ShapeDtypeStruct((N, RW), jnp.float32),
    ),
)


def _mid_body(aggp_ref, dii_ref, dio_ref, b_ref, w_ref, out_ref):
    h = (aggp_ref[0, :N] + aggp_ref[1, :N]) * dii_ref[...] + b_ref[...]
    h = jnp.maximum(h, 0.0) * dio_ref[...]
    out_ref[...] = jnp.dot(h, w_ref[...], preferred_element_type=jnp.float32)


def _mid_call(fout):
    return pl.pallas_call(
        _mid_body,
        out_shape=jax.ShapeDtypeStruct((N, fout), jnp.float32),
    )


def _final_body(aggp_ref, dii_ref, dio_ref, b3_ref, cp_ref, w4_ref, b4_ref,
                out_ref):
    x4 = (aggp_ref[0, :N] + aggp_ref[1, :N]) * dii_ref[...] + b3_ref[...]
    x4 = jnp.maximum(x4, 0.0) * dio_ref[...]
    w = cp_ref[0, :N, :1] + cp_ref[1, :N, :1]
    u = jnp.sum(x4 * w, axis=0, keepdims=True)
    out_ref[...] = (jnp.dot(u, w4_ref[...], preferred_element_type=jnp.float32)
                    * (1.0 / N) + b4_ref[...])


_final_call = pl.pallas_call(
    _final_body,
    out_shape=jax.ShapeDtypeStruct((1, 32), jnp.float32),
)


@jax.jit
def kernel(x, edge_index, W1, b1, W2, b2, W3, b3, W4, b4):
    ei3 = edge_index.astype(jnp.int32).reshape(2, E // CH, CH)

    dop, dip = _degrees_kernel(ei3)
    h1, dio, dii, dii16 = _first_call(x, W1, dop, dip)
    agg1, = _agg1_kernel(h1, ei3)
    h2 = _mid_call(64)(agg1, dii, dio, b1.reshape(1, -1), W2)
    agg2, = _agg2_kernel(h2, ei3)
    h3 = _mid_call(32)(agg2, dii, dio, b2.reshape(1, -1), W3)
    agg3, cp = _agg3_kernel(h3, ei3, dii16)
    out = _final_call(agg3, dii, dio, b3.reshape(1, -1),
                      cp, W4, b4.reshape(1, -1))
    return out


# speedup vs baseline: 17.0022x; 1.0669x over previous
import jax
import jax.numpy as jnp
from jax import lax
from jax.experimental import pallas as pl
from jax.experimental.pallas import tpu as pltpu
from jax.experimental.pallas import tpu_sc as plsc

N = 10000
NP = 10240
E = 320000
NC = 2
NS = 16
NW = NC * NS
EPW = E // NW
CH = 125
ROWS_PER_W = EPW // CH
GRP = 16
NG = ROWS_PER_W // GRP
AGRP = 40
ANG = ROWS_PER_W // AGRP
RW = 16

_mesh = plsc.VectorSubcoreMesh(core_axis_name="c", subcore_axis_name="s")
_sc_params = pltpu.CompilerParams(use_tc_tiling_on_sc=False)


def _degrees_body(ei3, do_out, di_out, sbuf, dbuf, obuf, zc, ssem, dsem, zsem,
                  sh_do, sh_di):
    c = lax.axis_index("c")
    s = lax.axis_index("s")
    wid = s * NC + c
    ones16 = jnp.full((16,), 1.0, jnp.float32)
    zv = jnp.zeros((16,), jnp.float32)

    def fill(i, _):
        obuf[i, :] = ones16
        return 0

    lax.fori_loop(0, CH, fill, 0)
    for r in range(16):
        zc[r, :] = zv

    def zbatch(i, _):
        for t in range(4):
            g = (s * 40 + i * 4 + t) * 16
            pltpu.async_copy(zc, sh_do.at[pl.ds(g, 16)], zsem)
            pltpu.async_copy(zc, sh_di.at[pl.ds(g, 16)], zsem)
        for t in range(4):
            g = (s * 40 + i * 4 + t) * 16
            pltpu.make_async_copy(zc, sh_do.at[pl.ds(g, 16)], zsem).wait()
            pltpu.make_async_copy(zc, sh_di.at[pl.ds(g, 16)], zsem).wait()
        return 0

    lax.fori_loop(0, (NP // 16) // NS // 4, zbatch, 0)
    plsc.subcore_barrier()

    row0 = wid * ROWS_PER_W
    for g in range(NG):
        pltpu.sync_copy(ei3.at[0, pl.ds(row0 + g * GRP, GRP)], sbuf)
        pltpu.sync_copy(ei3.at[1, pl.ds(row0 + g * GRP, GRP)], dbuf)

        def batch(i, _):
            for t in range(4):
                j = i * 4 + t
                pltpu.async_copy(obuf, sh_do.at[sbuf.at[j]], ssem, add=True)
                pltpu.async_copy(obuf, sh_di.at[dbuf.at[j]], dsem, add=True)
            for t in range(4):
                j = i * 4 + t
                pltpu.make_async_copy(obuf, sh_do.at[sbuf.at[j]],
                                      ssem).wait()
                pltpu.make_async_copy(obuf, sh_di.at[dbuf.at[j]],
                                      dsem).wait()
            return 0

        lax.fori_loop(0, GRP // 4, batch, 0)
    plsc.subcore_barrier()

    @pl.when(s == 0)
    def _():
        pltpu.sync_copy(sh_do, do_out.at[c])
        pltpu.sync_copy(sh_di, di_out.at[c])


_DEG_OUT_TYPE = (
    jax.ShapeDtypeStruct((NC, NP, RW), jnp.float32),
    jax.ShapeDtypeStruct((NC, NP, RW), jnp.float32),
)
_DEG_SCRATCH = (
    pltpu.VMEM((GRP, CH), jnp.int32),
    pltpu.VMEM((GRP, CH), jnp.int32),
    pltpu.VMEM((CH, RW), jnp.float32),
    pltpu.VMEM((16, RW), jnp.float32),
    pltpu.SemaphoreType.DMA,
    pltpu.SemaphoreType.DMA,
    pltpu.SemaphoreType.DMA,
    pltpu.VMEM_SHARED((NP, RW), jnp.float32),
    pltpu.VMEM_SHARED((NP, RW), jnp.float32),
)
_degrees_kernel = pl.kernel(
    _degrees_body, out_type=_DEG_OUT_TYPE, mesh=_mesh,
    scratch_types=_DEG_SCRATCH, compiler_params=_sc_params)


def _make_agg_parts(F, with_c, dt):
    out_type = [jax.ShapeDtypeStruct((NC, NP, F), dt)]
    scratch = [
        pltpu.VMEM((AGRP, CH), jnp.int32),
        pltpu.VMEM((AGRP, CH), jnp.int32),
        pltpu.VMEM((CH, F), dt),
        pltpu.VMEM((CH, F), dt),
        pltpu.VMEM((16, F), dt),
        pltpu.SemaphoreType.DMA,
        pltpu.SemaphoreType.DMA,
        pltpu.SemaphoreType.DMA,
        pltpu.VMEM_SHARED((NP, F), dt),
    ]
    if with_c:
        out_type.append(jax.ShapeDtypeStruct((NC, NP, RW), jnp.float32))
        scratch += [
            pltpu.VMEM((CH, RW), jnp.float32),
            pltpu.VMEM((CH, RW), jnp.float32),
            pltpu.VMEM((16, RW), jnp.float32),
            pltpu.SemaphoreType.DMA,
            pltpu.SemaphoreType.DMA,
            pltpu.VMEM_SHARED((NP, RW), jnp.float32),
        ]

    def body(*args):
        if with_c:
            (h, ei3, dii, agg_out, c_out,
             sbuf, dbuf, rows0, rows1, zbuf, sem0, sem1, zsem, sh_agg,
             vals0, vals1, zc, vsem0, vsem1, sh_c) = args
            vals = (vals0, vals1)
            vsem = (vsem0, vsem1)
        else:
            (h, ei3, agg_out,
             sbuf, dbuf, rows0, rows1, zbuf, sem0, sem1, zsem, sh_agg) = args
        rows = (rows0, rows1)
        sem = (sem0, sem1)
        c = lax.axis_index("c")
        s = lax.axis_index("s")
        wid = s * NC + c
        zlanes = 16 if dt == jnp.float32 else 32
        zvd = jnp.zeros((zlanes,), dt)
        for r in range(16):
            for q in range(F // zlanes):
                zbuf[r, pl.ds(q * zlanes, zlanes)] = zvd
        if with_c:
            zv = jnp.zeros((16,), jnp.float32)
            for r in range(16):
                zc[r, :] = zv

        def zbatch(i, _):
            for t in range(4):
                g = (s * 40 + i * 4 + t) * 16
                pltpu.async_copy(zbuf, sh_agg.at[pl.ds(g, 16)], zsem)
                if with_c:
                    pltpu.async_copy(zc, sh_c.at[pl.ds(g, 16)], zsem)
            for t in range(4):
                g = (s * 40 + i * 4 + t) * 16
                pltpu.make_async_copy(zbuf, sh_agg.at[pl.ds(g, 16)],
                                      zsem).wait()
                if with_c:
                    pltpu.make_async_copy(zc, sh_c.at[pl.ds(g, 16)],
                                          zsem).wait()
            return 0

        lax.fori_loop(0, (NP // 16) // NS // 4, zbatch, 0)
        plsc.subcore_barrier()

        def gathers(j, b):
            pltpu.async_copy(h.at[sbuf.at[j]], rows[b], sem[b])
            if with_c:
                pltpu.async_copy(dii.at[dbuf.at[j]], vals[b], vsem[b])

        def drain(j, b):
            pltpu.make_async_copy(h.at[sbuf.at[j]], rows[b], sem[b]).wait()
            pltpu.sync_copy(rows[b], sh_agg.at[dbuf.at[j]], add=True)
            if with_c:
                pltpu.make_async_copy(dii.at[dbuf.at[j]], vals[b],
                                      vsem[b]).wait()
                pltpu.sync_copy(vals[b], sh_c.at[sbuf.at[j]], add=True)

        def chunk2(i, _):
            j0 = 2 * i
            gathers(j0 + 1, 1)
            drain(j0, 0)

            @pl.when(j0 + 2 < AGRP)
            def _():
                gathers(j0 + 2, 0)

            drain(j0 + 1, 1)
            return 0

        row0 = wid * ROWS_PER_W
        for g in range(ANG):
            pltpu.sync_copy(ei3.at[0, pl.ds(row0 + g * AGRP, AGRP)], sbuf)
            pltpu.sync_copy(ei3.at[1, pl.ds(row0 + g * AGRP, AGRP)], dbuf)
            gathers(0, 0)
            lax.fori_loop(0, AGRP // 2, chunk2, 0)
        plsc.subcore_barrier()
        rpt = NP // NS
        pltpu.sync_copy(sh_agg.at[pl.ds(s * rpt, rpt)],
                        agg_out.at[c, pl.ds(s * rpt, rpt)])
        if with_c:
            @pl.when(s == 0)
            def _():
                pltpu.sync_copy(sh_c, c_out.at[c])

    return body, tuple(out_type), tuple(scratch)


def _make_agg_kernel(F, with_c, dt=jnp.float32):
    body, out_type, scratch = _make_agg_parts(F, with_c, dt)
    return pl.kernel(body, out_type=out_type, mesh=_mesh,
                     scratch_types=scratch, compiler_params=_sc_params)


_agg1_kernel = _make_agg_kernel(128, with_c=False, dt=jnp.bfloat16)
_agg2_kernel = _make_agg_kernel(64, with_c=False, dt=jnp.bfloat16)
_agg3_kernel = _make_agg_kernel(32, with_c=True)


def _first_body(x_ref, w_ref, dop_ref, dip_ref, h_ref, dio_ref, dii_ref,
                dii16_ref):
    deg_o = jnp.maximum(dop_ref[0, :N, :1] + dop_ref[1, :N, :1], 1.0)
    deg_i = jnp.maximum(dip_ref[0, :N, :1] + dip_ref[1, :N, :1], 1.0)
    dio = lax.rsqrt(deg_o)
    dii = lax.rsqrt(deg_i)
    dio_ref[...] = dio
    dii_ref[...] = dii
    dii16_ref[...] = jnp.broadcast_to(dii, (N, RW))
    h_ref[...] = jnp.dot(x_ref[...] * dio, w_ref[...],
                         preferred_element_type=jnp.float32
                         ).astype(h_ref.dtype)


_first_call = pl.pallas_call(
    _first_body,
    out_shape=(
        jax.ShapeDtypeStruct((N, 128), jnp.bfloat16),
        jax.ShapeDtypeStruct((N, 1), jnp.float32),
        jax.ShapeDtypeStruct((N, 1), jnp.float32),
        jax.ShapeDtypeStruct((N, RW), jnp.float32),
    ),
)


def _mid_body(aggp_ref, dii_ref, dio_ref, b_ref, w_ref, out_ref):
    agg = (aggp_ref[0, :N].astype(jnp.float32)
           + aggp_ref[1, :N].astype(jnp.float32))
    h = agg * dii_ref[...] + b_ref[...]
    h = jnp.maximum(h, 0.0) * dio_ref[...]
    out_ref[...] = jnp.dot(h, w_ref[...], preferred_element_type=jnp.float32
                           ).astype(out_ref.dtype)


def _mid_call(fout, dt=jnp.float32):
    return pl.pallas_call(
        _mid_body,
        out_shape=jax.ShapeDtypeStruct((N, fout), dt),
    )


def _final_body(aggp_ref, dii_ref, dio_ref, b3_ref, cp_ref, w4_ref, b4_ref,
                out_ref):
    x4 = (aggp_ref[0, :N].astype(jnp.float32)
          + aggp_ref[1, :N].astype(jnp.float32)) * dii_ref[...] + b3_ref[...]
    x4 = jnp.maximum(x4, 0.0) * dio_ref[...]
    w = cp_ref[0, :N, :1] + cp_ref[1, :N, :1]
    u = jnp.sum(x4 * w, axis=0, keepdims=True)
    out_ref[...] = (jnp.dot(u, w4_ref[...], preferred_element_type=jnp.float32)
                    * (1.0 / N) + b4_ref[...])


_final_call = pl.pallas_call(
    _final_body,
    out_shape=jax.ShapeDtypeStruct((1, 32), jnp.float32),
)


@jax.jit
def kernel(x, edge_index, W1, b1, W2, b2, W3, b3, W4, b4):
    ei3 = edge_index.astype(jnp.int32).reshape(2, E // CH, CH)

    dop, dip = _degrees_kernel(ei3)
    h1, dio, dii, dii16 = _first_call(x, W1, dop, dip)
    agg1, = _agg1_kernel(h1, ei3)
    h2 = _mid_call(64, jnp.bfloat16)(agg1, dii, dio, b1.reshape(1, -1), W2)
    agg2, = _agg2_kernel(h2, ei3)
    h3 = _mid_call(32)(agg2, dii, dio, b2.reshape(1, -1), W3)
    agg3, cp = _agg3_kernel(h3, ei3, dii16)
    out = _final_call(agg3, dii, dio, b3.reshape(1, -1),
                      cp, W4, b4.reshape(1, -1))
    return out


# speedup vs baseline: 17.2175x; 1.0127x over previous
import jax
import jax.numpy as jnp
from jax import lax
from jax.experimental import pallas as pl
from jax.experimental.pallas import tpu as pltpu
from jax.experimental.pallas import tpu_sc as plsc

N = 10000
NP = 10240
E = 320000
NC = 2
NS = 16
NW = NC * NS
EPW = E // NW
CH = 125
ROWS_PER_W = EPW // CH
GRP = 16
NG = ROWS_PER_W // GRP
AGRP = 40
ANG = ROWS_PER_W // AGRP
RW = 16

_mesh = plsc.VectorSubcoreMesh(core_axis_name="c", subcore_axis_name="s")
_sc_params = pltpu.CompilerParams(use_tc_tiling_on_sc=False)


def _degrees_body(ei3, do_out, di_out, sbuf, dbuf, obuf, zc, ssem, dsem, zsem,
                  sh_do, sh_di):
    c = lax.axis_index("c")
    s = lax.axis_index("s")
    wid = s * NC + c
    ones16 = jnp.full((16,), 1.0, jnp.float32)
    zv = jnp.zeros((16,), jnp.float32)

    def fill(i, _):
        obuf[i, :] = ones16
        return 0

    lax.fori_loop(0, CH, fill, 0)
    for r in range(16):
        zc[r, :] = zv

    def zbatch(i, _):
        for t in range(4):
            g = (s * 40 + i * 4 + t) * 16
            pltpu.async_copy(zc, sh_do.at[pl.ds(g, 16)], zsem)
            pltpu.async_copy(zc, sh_di.at[pl.ds(g, 16)], zsem)
        for t in range(4):
            g = (s * 40 + i * 4 + t) * 16
            pltpu.make_async_copy(zc, sh_do.at[pl.ds(g, 16)], zsem).wait()
            pltpu.make_async_copy(zc, sh_di.at[pl.ds(g, 16)], zsem).wait()
        return 0

    lax.fori_loop(0, (NP // 16) // NS // 4, zbatch, 0)
    plsc.subcore_barrier()

    row0 = wid * ROWS_PER_W
    for g in range(NG):
        pltpu.sync_copy(ei3.at[0, pl.ds(row0 + g * GRP, GRP)], sbuf)
        pltpu.sync_copy(ei3.at[1, pl.ds(row0 + g * GRP, GRP)], dbuf)

        def batch(i, _):
            for t in range(4):
                j = i * 4 + t
                pltpu.async_copy(obuf, sh_do.at[sbuf.at[j]], ssem, add=True)
                pltpu.async_copy(obuf, sh_di.at[dbuf.at[j]], dsem, add=True)
            for t in range(4):
                j = i * 4 + t
                pltpu.make_async_copy(obuf, sh_do.at[sbuf.at[j]],
                                      ssem).wait()
                pltpu.make_async_copy(obuf, sh_di.at[dbuf.at[j]],
                                      dsem).wait()
            return 0

        lax.fori_loop(0, GRP // 4, batch, 0)
    plsc.subcore_barrier()

    @pl.when(s == 0)
    def _():
        pltpu.sync_copy(sh_do, do_out.at[c])
        pltpu.sync_copy(sh_di, di_out.at[c])


_DEG_OUT_TYPE = (
    jax.ShapeDtypeStruct((NC, NP, RW), jnp.float32),
    jax.ShapeDtypeStruct((NC, NP, RW), jnp.float32),
)
_DEG_SCRATCH = (
    pltpu.VMEM((GRP, CH), jnp.int32),
    pltpu.VMEM((GRP, CH), jnp.int32),
    pltpu.VMEM((CH, RW), jnp.float32),
    pltpu.VMEM((16, RW), jnp.float32),
    pltpu.SemaphoreType.DMA,
    pltpu.SemaphoreType.DMA,
    pltpu.SemaphoreType.DMA,
    pltpu.VMEM_SHARED((NP, RW), jnp.float32),
    pltpu.VMEM_SHARED((NP, RW), jnp.float32),
)
_degrees_kernel = pl.kernel(
    _degrees_body, out_type=_DEG_OUT_TYPE, mesh=_mesh,
    scratch_types=_DEG_SCRATCH, compiler_params=_sc_params)


def _make_agg_parts(F, with_c, dt):
    out_type = [jax.ShapeDtypeStruct((NC, NP, F), dt)]
    scratch = [
        pltpu.VMEM((AGRP, CH), jnp.int32),
        pltpu.VMEM((AGRP, CH), jnp.int32),
        pltpu.VMEM((CH, F), dt),
        pltpu.VMEM((CH, F), dt),
        pltpu.VMEM((16, F), dt),
        pltpu.SemaphoreType.DMA,
        pltpu.SemaphoreType.DMA,
        pltpu.SemaphoreType.DMA,
        pltpu.VMEM_SHARED((NP, F), dt),
    ]
    if with_c:
        out_type.append(jax.ShapeDtypeStruct((NC, NP, RW), jnp.float32))
        scratch += [
            pltpu.VMEM((CH, RW), jnp.float32),
            pltpu.VMEM((CH, RW), jnp.float32),
            pltpu.VMEM((16, RW), jnp.float32),
            pltpu.SemaphoreType.DMA,
            pltpu.SemaphoreType.DMA,
            pltpu.VMEM_SHARED((NP, RW), jnp.float32),
        ]

    def body(*args):
        if with_c:
            (h, ei3, dii, agg_out, c_out,
             sbuf, dbuf, rows0, rows1, zbuf, sem0, sem1, zsem, sh_agg,
             vals0, vals1, zc, vsem0, vsem1, sh_c) = args
            vals = (vals0, vals1)
            vsem = (vsem0, vsem1)
        else:
            (h, ei3, agg_out,
             sbuf, dbuf, rows0, rows1, zbuf, sem0, sem1, zsem, sh_agg) = args
        rows = (rows0, rows1)
        sem = (sem0, sem1)
        c = lax.axis_index("c")
        s = lax.axis_index("s")
        wid = s * NC + c
        zlanes = 16 if dt == jnp.float32 else 32
        zvd = jnp.zeros((zlanes,), dt)
        for r in range(16):
            for q in range(F // zlanes):
                zbuf[r, pl.ds(q * zlanes, zlanes)] = zvd
        if with_c:
            zv = jnp.zeros((16,), jnp.float32)
            for r in range(16):
                zc[r, :] = zv

        def zbatch(i, _):
            for t in range(4):
                g = (s * 40 + i * 4 + t) * 16
                pltpu.async_copy(zbuf, sh_agg.at[pl.ds(g, 16)], zsem)
                if with_c:
                    pltpu.async_copy(zc, sh_c.at[pl.ds(g, 16)], zsem)
            for t in range(4):
                g = (s * 40 + i * 4 + t) * 16
                pltpu.make_async_copy(zbuf, sh_agg.at[pl.ds(g, 16)],
                                      zsem).wait()
                if with_c:
                    pltpu.make_async_copy(zc, sh_c.at[pl.ds(g, 16)],
                                          zsem).wait()
            return 0

        lax.fori_loop(0, (NP // 16) // NS // 4, zbatch, 0)
        plsc.subcore_barrier()

        def gathers(j, b):
            pltpu.async_copy(h.at[sbuf.at[j]], rows[b], sem[b])
            if with_c:
                pltpu.async_copy(dii.at[dbuf.at[j]], vals[b], vsem[b])

        def drain(j, b):
            pltpu.make_async_copy(h.at[sbuf.at[j]], rows[b], sem[b]).wait()
            pltpu.sync_copy(rows[b], sh_agg.at[dbuf.at[j]], add=True)
            if with_c:
                pltpu.make_async_copy(dii.at[dbuf.at[j]], vals[b],
                                      vsem[b]).wait()
                pltpu.sync_copy(vals[b], sh_c.at[sbuf.at[j]], add=True)

        def chunk2(i, _):
            j0 = 2 * i
            gathers(j0 + 1, 1)
            drain(j0, 0)

            @pl.when(j0 + 2 < AGRP)
            def _():
                gathers(j0 + 2, 0)

            drain(j0 + 1, 1)
            return 0

        row0 = wid * ROWS_PER_W
        for g in range(ANG):
            pltpu.sync_copy(ei3.at[0, pl.ds(row0 + g * AGRP, AGRP)], sbuf)
            pltpu.sync_copy(ei3.at[1, pl.ds(row0 + g * AGRP, AGRP)], dbuf)
            gathers(0, 0)
            lax.fori_loop(0, AGRP // 2, chunk2, 0)
        plsc.subcore_barrier()
        rpt = NP // NS
        pltpu.sync_copy(sh_agg.at[pl.ds(s * rpt, rpt)],
                        agg_out.at[c, pl.ds(s * rpt, rpt)])
        if with_c:
            @pl.when(s == 0)
            def _():
                pltpu.sync_copy(sh_c, c_out.at[c])

    return body, tuple(out_type), tuple(scratch)


def _make_agg_kernel(F, with_c, dt=jnp.float32):
    body, out_type, scratch = _make_agg_parts(F, with_c, dt)
    return pl.kernel(body, out_type=out_type, mesh=_mesh,
                     scratch_types=scratch, compiler_params=_sc_params)


_agg1_kernel = _make_agg_kernel(128, with_c=False, dt=jnp.bfloat16)
_agg2_kernel = _make_agg_kernel(64, with_c=False, dt=jnp.bfloat16)
_agg3_kernel = _make_agg_kernel(32, with_c=True, dt=jnp.bfloat16)


def _first_body(x_ref, w_ref, dop_ref, dip_ref, h_ref, dio_ref, dii_ref,
                dii16_ref):
    deg_o = jnp.maximum(dop_ref[0, :N, :1] + dop_ref[1, :N, :1], 1.0)
    deg_i = jnp.maximum(dip_ref[0, :N, :1] + dip_ref[1, :N, :1], 1.0)
    dio = lax.rsqrt(deg_o)
    dii = lax.rsqrt(deg_i)
    dio_ref[...] = dio
    dii_ref[...] = dii
    dii16_ref[...] = jnp.broadcast_to(dii, (N, RW))
    h_ref[...] = jnp.dot(x_ref[...] * dio, w_ref[...],
                         preferred_element_type=jnp.float32
                         ).astype(h_ref.dtype)


_first_call = pl.pallas_call(
    _first_body,
    out_shape=(
        jax.ShapeDtypeStruct((N, 128), jnp.bfloat16),
        jax.ShapeDtypeStruct((N, 1), jnp.float32),
        jax.ShapeDtypeStruct((N, 1), jnp.float32),
        jax.ShapeDtypeStruct((N, RW), jnp.float32),
    ),
)


def _mid_body(aggp_ref, dii_ref, dio_ref, b_ref, w_ref, out_ref):
    agg = (aggp_ref[0, :N].astype(jnp.float32)
           + aggp_ref[1, :N].astype(jnp.float32))
    h = agg * dii_ref[...] + b_ref[...]
    h = jnp.maximum(h, 0.0) * dio_ref[...]
    out_ref[...] = jnp.dot(h, w_ref[...], preferred_element_type=jnp.float32
                           ).astype(out_ref.dtype)


def _mid_call(fout, dt=jnp.float32):
    return pl.pallas_call(
        _mid_body,
        out_shape=jax.ShapeDtypeStruct((N, fout), dt),
    )


def _final_body(aggp_ref, dii_ref, dio_ref, b3_ref, cp_ref, w4_ref, b4_ref,
                out_ref):
    x4 = (aggp_ref[0, :N].astype(jnp.float32)
          + aggp_ref[1, :N].astype(jnp.float32)) * dii_ref[...] + b3_ref[...]
    x4 = jnp.maximum(x4, 0.0) * dio_ref[...]
    w = cp_ref[0, :N, :1] + cp_ref[1, :N, :1]
    u = jnp.sum(x4 * w, axis=0, keepdims=True)
    out_ref[...] = (jnp.dot(u, w4_ref[...], preferred_element_type=jnp.float32)
                    * (1.0 / N) + b4_ref[...])


_final_call = pl.pallas_call(
    _final_body,
    out_shape=jax.ShapeDtypeStruct((1, 32), jnp.float32),
)


@jax.jit
def kernel(x, edge_index, W1, b1, W2, b2, W3, b3, W4, b4):
    ei3 = edge_index.astype(jnp.int32).reshape(2, E // CH, CH)

    dop, dip = _degrees_kernel(ei3)
    h1, dio, dii, dii16 = _first_call(x, W1, dop, dip)
    agg1, = _agg1_kernel(h1, ei3)
    h2 = _mid_call(64, jnp.bfloat16)(agg1, dii, dio, b1.reshape(1, -1), W2)
    agg2, = _agg2_kernel(h2, ei3)
    h3 = _mid_call(32, jnp.bfloat16)(agg2, dii, dio, b2.reshape(1, -1), W3)
    agg3, cp = _agg3_kernel(h3, ei3, dii16)
    out = _final_call(agg3, dii, dio, b3.reshape(1, -1),
                      cp, W4, b4.reshape(1, -1))
    return out
